# baseline (speedup 1.0000x reference)
import jax
import jax.numpy as jnp
from jax import lax
from jax.experimental import pallas as pl
from jax.experimental.pallas import tpu as pltpu
from jax.experimental.pallas import tpu_sc as plsc

BATCH = 4096
HIST = 50
EMBED = 64
B = BATCH * HIST
ROW = 128
NROWS = B // ROW
ROWS_PER_H = BATCH // ROW
NUM_CORES = 2
NUM_SUBCORES = 16
NUM_WORKERS = NUM_CORES * NUM_SUBCORES
ROWS_PER_W = NROWS // NUM_WORKERS


def _gather_body(table_hbm, idx_hbm, out_hbm,
                 idx0, idx1, rows0, rows1, tb0, tb1, gsem0, gsem1):
    wid = lax.axis_index("s") * NUM_CORES + lax.axis_index("c")
    r0 = wid * ROWS_PER_W
    bufs = ((idx0, rows0, tb0, gsem0), (idx1, rows1, tb1, gsem1))

    def stage(r, b):
        idx_v, rows_v, _, gsem = bufs[b]
        pltpu.sync_copy(idx_hbm.at[r], idx_v)
        pltpu.async_copy(table_hbm.at[idx_v], rows_v, gsem)

    def finish(r, b):
        idx_v, rows_v, tbuf, gsem = bufs[b]
        pltpu.make_async_copy(table_hbm.at[idx_v], rows_v, gsem).wait()
        iota = lax.iota(jnp.int32, 16)
        rowvecs = tuple(iota + 16 * j for j in range(8))

        def dbody(d, rv):
            dsplat = jnp.full((16,), d, jnp.int32)
            for j in range(8):
                v = plsc.load_gather(rows_v, [rv[j], dsplat])
                tbuf[d, pl.ds(16 * j, 16)] = v
            return rv

        lax.fori_loop(0, EMBED, dbody, rowvecs)
        h = r // ROWS_PER_H
        b0 = (r % ROWS_PER_H) * ROW
        pltpu.sync_copy(tbuf, out_hbm.at[h, :, pl.ds(b0, ROW)])

    stage(r0, 0)

    def body(i, carry):
        r = r0 + 2 * i
        stage(r + 1, 1)
        finish(r, 0)

        @pl.when(i < ROWS_PER_W // 2 - 1)
        def _():
            stage(r + 2, 0)

        finish(r + 1, 1)
        return carry

    lax.fori_loop(0, ROWS_PER_W // 2, body, 0)


@jax.jit
def kernel(x, table):
    idxT = x.T.reshape(NROWS, ROW).astype(jnp.int32)
    mesh = plsc.VectorSubcoreMesh(core_axis_name="c", subcore_axis_name="s")
    outT = pl.kernel(
        _gather_body,
        mesh=mesh,
        out_type=jax.ShapeDtypeStruct((HIST, EMBED, BATCH), jnp.float32),
        scratch_types=[
            pltpu.VMEM((ROW,), jnp.int32),
            pltpu.VMEM((ROW,), jnp.int32),
            pltpu.VMEM((ROW, EMBED), jnp.float32),
            pltpu.VMEM((ROW, EMBED), jnp.float32),
            pltpu.VMEM((EMBED, ROW), jnp.float32),
            pltpu.VMEM((EMBED, ROW), jnp.float32),
            pltpu.SemaphoreType.DMA,
            pltpu.SemaphoreType.DMA,
        ],
        compiler_params=pltpu.CompilerParams(
            use_tc_tiling_on_sc=False, needs_layout_passes=False),
    )(table, idxT)
    return outT.transpose(2, 0, 1)


# speedup vs baseline: 1.0500x; 1.0500x over previous
import jax
import jax.numpy as jnp
from jax import lax
from jax.experimental import pallas as pl
from jax.experimental.pallas import tpu as pltpu
from jax.experimental.pallas import tpu_sc as plsc

BATCH = 4096
HIST = 50
EMBED = 64
B = BATCH * HIST
ROW = 128
NROWS = B // ROW
ROWS_PER_H = BATCH // ROW
NUM_CORES = 2
NUM_SUBCORES = 16
NUM_WORKERS = NUM_CORES * NUM_SUBCORES
ROWS_PER_W = NROWS // NUM_WORKERS


def _gather_body(table_hbm, idx_hbm, out_hbm,
                 idx_all, rows0, rows1, tb0, tb1, gsem0, gsem1):
    wid = lax.axis_index("s") * NUM_CORES + lax.axis_index("c")
    r0 = wid * ROWS_PER_W
    bufs = ((rows0, tb0, gsem0), (rows1, tb1, gsem1))

    pltpu.sync_copy(idx_hbm.at[pl.ds(r0, ROWS_PER_W)], idx_all)

    def stage(i_local, b):
        rows_v, _, gsem = bufs[b]
        pltpu.async_copy(table_hbm.at[idx_all.at[i_local]], rows_v, gsem)

    def finish(i_local, b):
        rows_v, tbuf, gsem = bufs[b]
        pltpu.make_async_copy(
            table_hbm.at[idx_all.at[i_local]], rows_v, gsem).wait()
        iota = lax.iota(jnp.int32, 16)

        def jbody(j, io):
            rowvec = io + j * 16
            for d in range(EMBED):
                v = plsc.load_gather(
                    rows_v, [rowvec, jnp.full((16,), d, jnp.int32)])
                tbuf[d, pl.ds(j * 16, 16)] = v
            return io

        lax.fori_loop(0, ROW // 16, jbody, iota)
        r = r0 + i_local
        h = r // ROWS_PER_H
        b0 = (r % ROWS_PER_H) * ROW
        pltpu.sync_copy(tbuf, out_hbm.at[h, :, pl.ds(b0, ROW)])

    stage(0, 0)

    def body(i, carry):
        il = 2 * i
        stage(il + 1, 1)
        finish(il, 0)

        @pl.when(i < ROWS_PER_W // 2 - 1)
        def _():
            stage(il + 2, 0)

        finish(il + 1, 1)
        return carry

    lax.fori_loop(0, ROWS_PER_W // 2, body, 0)


@jax.jit
def kernel(x, table):
    idxT = x.T.reshape(NROWS, ROW).astype(jnp.int32)
    mesh = plsc.VectorSubcoreMesh(core_axis_name="c", subcore_axis_name="s")
    outT = pl.kernel(
        _gather_body,
        mesh=mesh,
        out_type=jax.ShapeDtypeStruct((HIST, EMBED, BATCH), jnp.float32),
        scratch_types=[
            pltpu.VMEM((ROWS_PER_W, ROW), jnp.int32),
            pltpu.VMEM((ROW, EMBED), jnp.float32),
            pltpu.VMEM((ROW, EMBED), jnp.float32),
            pltpu.VMEM((EMBED, ROW), jnp.float32),
            pltpu.VMEM((EMBED, ROW), jnp.float32),
            pltpu.SemaphoreType.DMA,
            pltpu.SemaphoreType.DMA,
        ],
        compiler_params=pltpu.CompilerParams(
            use_tc_tiling_on_sc=False, needs_layout_passes=False),
    )(table, idxT)
    return outT.transpose(2, 0, 1)


# speedup vs baseline: 2.6297x; 2.5044x over previous
import jax
import jax.numpy as jnp
from jax import lax
from jax.experimental import pallas as pl
from jax.experimental.pallas import tpu as pltpu
from jax.experimental.pallas import tpu_sc as plsc

BATCH = 4096
HIST = 50
EMBED = 64
B = BATCH * HIST
ROW = 128
NROWS = B // ROW
ROWS_PER_H = BATCH // ROW
NUM_CORES = 2
NUM_SUBCORES = 16
NUM_WORKERS = NUM_CORES * NUM_SUBCORES
ROWS_PER_W = NROWS // NUM_WORKERS


def _gather_body(table_hbm, idx_hbm, out_hbm,
                 idx_all, rows0, rows1, tb0, tb1, gsem0, gsem1):
    wid = lax.axis_index("s") * NUM_CORES + lax.axis_index("c")
    r0 = wid * ROWS_PER_W
    bufs = ((rows0, tb0, gsem0), (rows1, tb1, gsem1))

    pltpu.sync_copy(idx_hbm.at[pl.ds(r0, ROWS_PER_W)], idx_all)

    def stage(i_local, b):
        rows_v, _, gsem = bufs[b]
        pltpu.async_copy(table_hbm.at[idx_all.at[i_local]], rows_v, gsem)

    iota = lax.iota(jnp.int32, 16)
    dvecs = tuple(iota + 16 * k for k in range(EMBED // 16))

    def finish(i_local, b):
        rows_v, tbuf, gsem = bufs[b]
        pltpu.make_async_copy(
            table_hbm.at[idx_all.at[i_local]], rows_v, gsem).wait()

        @plsc.parallel_loop(0, ROW, step=1, unroll=4)
        def _(l):
            lsplat = jnp.full((16,), l, jnp.int32)
            for k in range(EMBED // 16):
                v = rows_v[l, pl.ds(16 * k, 16)]
                plsc.store_scatter(tbuf, [dvecs[k], lsplat], v)

        r = r0 + i_local
        h = r // ROWS_PER_H
        b0 = (r % ROWS_PER_H) * ROW
        pltpu.sync_copy(tbuf.at[:, pl.ds(0, ROW)],
                        out_hbm.at[h, :, pl.ds(b0, ROW)])

    stage(0, 0)

    def body(i, carry):
        il = 2 * i
        stage(il + 1, 1)
        finish(il, 0)

        @pl.when(i < ROWS_PER_W // 2 - 1)
        def _():
            stage(il + 2, 0)

        finish(il + 1, 1)
        return carry

    lax.fori_loop(0, ROWS_PER_W // 2, body, 0)


@jax.jit
def kernel(x, table):
    idxT = x.T.reshape(NROWS, ROW).astype(jnp.int32)
    mesh = plsc.VectorSubcoreMesh(core_axis_name="c", subcore_axis_name="s")
    outT = pl.kernel(
        _gather_body,
        mesh=mesh,
        out_type=jax.ShapeDtypeStruct((HIST, EMBED, BATCH), jnp.float32),
        scratch_types=[
            pltpu.VMEM((ROWS_PER_W, ROW), jnp.int32),
            pltpu.VMEM((ROW, EMBED), jnp.float32),
            pltpu.VMEM((ROW, EMBED), jnp.float32),
            pltpu.VMEM((EMBED, ROW + 1), jnp.float32),
            pltpu.VMEM((EMBED, ROW + 1), jnp.float32),
            pltpu.SemaphoreType.DMA,
            pltpu.SemaphoreType.DMA,
        ],
        compiler_params=pltpu.CompilerParams(
            use_tc_tiling_on_sc=False, needs_layout_passes=False),
    )(table, idxT)
    return outT.transpose(2, 0, 1)


# speedup vs baseline: 3.3645x; 1.2794x over previous
import jax
import jax.numpy as jnp
from jax import lax
from jax.experimental import pallas as pl
from jax.experimental.pallas import tpu as pltpu
from jax.experimental.pallas import tpu_sc as plsc

BATCH = 4096
HIST = 50
EMBED = 64
B = BATCH * HIST
ROW = 128
NROWS = B // ROW
ROWS_PER_H = BATCH // ROW
NUM_CORES = 2
NUM_SUBCORES = 16
NUM_WORKERS = NUM_CORES * NUM_SUBCORES
ROWS_PER_W = NROWS // NUM_WORKERS
TPAD = 2 * EMBED
PITCH = ROW + 1


def _gather_body(table_hbm, idx_hbm, out_hbm,
                 idx_all, rows0, rows1, tb0, tb1, gsem0, gsem1):
    wid = lax.axis_index("s") * NUM_CORES + lax.axis_index("c")
    r0 = wid * ROWS_PER_W
    bufs = ((rows0, tb0, gsem0), (rows1, tb1, gsem1))

    pltpu.sync_copy(idx_hbm.at[pl.ds(r0, ROWS_PER_W)], idx_all)

    def stage(i_local, b):
        rows_v, _, gsem = bufs[b]
        pltpu.async_copy(table_hbm.at[idx_all.at[i_local]], rows_v, gsem)

    iota = lax.iota(jnp.int32, 16)
    dvecs = tuple(iota + 16 * k for k in range(EMBED // 16))

    def finish(i_local, b):
        rows_v, tbuf, gsem = bufs[b]
        pltpu.make_async_copy(
            table_hbm.at[idx_all.at[i_local]], rows_v, gsem).wait()

        @plsc.parallel_loop(0, ROW, step=1, unroll=4)
        def _(l):
            lsplat = jnp.full((16,), l, jnp.int32)
            for k in range(EMBED // 16):
                v = rows_v[l, pl.ds(16 * k, 16)]
                plsc.store_scatter(tbuf, [dvecs[k], lsplat], v)

        r = r0 + i_local
        h = r // ROWS_PER_H
        c = r % ROWS_PER_H
        for t in range(EMBED // 8):
            pltpu.sync_copy(tbuf.at[pl.ds(8 * t, 8), pl.ds(0, ROW)],
                            out_hbm.at[h, t, c])

    stage(0, 0)

    def body(i, carry):
        il = 2 * i
        stage(il + 1, 1)
        finish(il, 0)

        @pl.when(i < ROWS_PER_W // 2 - 1)
        def _():
            stage(il + 2, 0)

        finish(il + 1, 1)
        return carry

    lax.fori_loop(0, ROWS_PER_W // 2, body, 0)


@jax.jit
def kernel(x, table):
    idxT = x.T.reshape(NROWS, ROW).astype(jnp.int32)
    tp = jnp.pad(table, ((0, 0), (0, TPAD - EMBED)))
    mesh = plsc.VectorSubcoreMesh(core_axis_name="c", subcore_axis_name="s")
    out5d = pl.kernel(
        _gather_body,
        mesh=mesh,
        out_type=jax.ShapeDtypeStruct(
            (HIST, EMBED // 8, ROWS_PER_H, 8, ROW), jnp.float32),
        scratch_types=[
            pltpu.VMEM((ROWS_PER_W, ROW), jnp.int32),
            pltpu.VMEM((ROW, TPAD), jnp.float32),
            pltpu.VMEM((ROW, TPAD), jnp.float32),
            pltpu.VMEM((EMBED, PITCH), jnp.float32),
            pltpu.VMEM((EMBED, PITCH), jnp.float32),
            pltpu.SemaphoreType.DMA,
            pltpu.SemaphoreType.DMA,
        ],
        compiler_params=pltpu.CompilerParams(
            use_tc_tiling_on_sc=False, needs_layout_passes=False),
    )(tp, idxT)
    return out5d.transpose(2, 4, 0, 1, 3).reshape(BATCH, HIST, EMBED)


# speedup vs baseline: 3.6828x; 1.0946x over previous
import jax
import jax.numpy as jnp
from jax import lax
from jax.experimental import pallas as pl
from jax.experimental.pallas import tpu as pltpu
from jax.experimental.pallas import tpu_sc as plsc

BATCH = 4096
HIST = 50
EMBED = 64
NP = 100000
B = BATCH * HIST
ROW = 128
NROWS = B // ROW
ROWS_PER_H = BATCH // ROW
NUM_CORES = 2
NUM_SUBCORES = 16
NUM_WORKERS = NUM_CORES * NUM_SUBCORES
ROWS_PER_W = NROWS // NUM_WORKERS
TPAD = 2 * EMBED
PITCH = ROW + 1


def _gather_body(table_hbm, idx_hbm, out_hbm,
                 idx_all, rows0, rows1, tb0, tb1, gsem0, gsem1):
    wid = lax.axis_index("s") * NUM_CORES + lax.axis_index("c")
    r0 = wid * ROWS_PER_W
    bufs = ((rows0, tb0, gsem0), (rows1, tb1, gsem1))

    pltpu.sync_copy(idx_hbm.at[pl.ds(r0, ROWS_PER_W)], idx_all)

    def stage(i_local, b):
        rows_v, _, gsem = bufs[b]
        pltpu.async_copy(table_hbm.at[idx_all.at[i_local]], rows_v, gsem)

    iota = lax.iota(jnp.int32, 16)
    dvecs = tuple(iota + 16 * k for k in range(EMBED // 16))

    def finish(i_local, b):
        rows_v, tbuf, gsem = bufs[b]
        pltpu.make_async_copy(
            table_hbm.at[idx_all.at[i_local]], rows_v, gsem).wait()

        @plsc.parallel_loop(0, ROW, step=1, unroll=4)
        def _(l):
            lsplat = jnp.full((16,), l, jnp.int32)
            for k in range(EMBED // 16):
                v = rows_v[l, pl.ds(16 * k, 16)]
                plsc.store_scatter(tbuf, [dvecs[k], lsplat], v)

        r = r0 + i_local
        h = r // ROWS_PER_H
        c = r % ROWS_PER_H
        for t in range(EMBED // 8):
            pltpu.sync_copy(tbuf.at[pl.ds(8 * t, 8), pl.ds(0, ROW)],
                            out_hbm.at[h, t, c])

    stage(0, 0)

    def body(i, carry):
        il = 2 * i
        stage(il + 1, 1)
        finish(il, 0)

        @pl.when(i < ROWS_PER_W // 2 - 1)
        def _():
            stage(il + 2, 0)

        finish(il + 1, 1)
        return carry

    lax.fori_loop(0, ROWS_PER_W // 2, body, 0)


@jax.jit
def kernel(x, table):
    idxT = (x.T.reshape(NROWS, ROW) * 2).astype(jnp.int32)
    tp = jnp.pad(table, ((0, 0), (0, TPAD - EMBED))).reshape(2 * NP, EMBED)
    mesh = plsc.VectorSubcoreMesh(core_axis_name="c", subcore_axis_name="s")
    out5d = pl.kernel(
        _gather_body,
        mesh=mesh,
        out_type=jax.ShapeDtypeStruct(
            (HIST, EMBED // 8, ROWS_PER_H, 8, ROW), jnp.float32),
        scratch_types=[
            pltpu.VMEM((ROWS_PER_W, ROW), jnp.int32),
            pltpu.VMEM((ROW, EMBED), jnp.float32),
            pltpu.VMEM((ROW, EMBED), jnp.float32),
            pltpu.VMEM((EMBED, PITCH), jnp.float32),
            pltpu.VMEM((EMBED, PITCH), jnp.float32),
            pltpu.SemaphoreType.DMA,
            pltpu.SemaphoreType.DMA,
        ],
        compiler_params=pltpu.CompilerParams(
            use_tc_tiling_on_sc=False, needs_layout_passes=False),
    )(tp, idxT)
    return out5d.transpose(2, 4, 0, 1, 3).reshape(BATCH, HIST, EMBED)


# speedup vs baseline: 4.0292x; 1.0941x over previous
import jax
import jax.numpy as jnp
from jax import lax
from jax.experimental import pallas as pl
from jax.experimental.pallas import tpu as pltpu
from jax.experimental.pallas import tpu_sc as plsc

BATCH = 4096
HIST = 50
EMBED = 64
NP = 100000
B = BATCH * HIST
ROW = 128
NROWS = B // ROW
ROWS_PER_H = BATCH // ROW
NUM_CORES = 2
NUM_SUBCORES = 16
NUM_WORKERS = NUM_CORES * NUM_SUBCORES
ROWS_PER_W = NROWS // NUM_WORKERS
TPAD = 2 * EMBED
PITCH = ROW + 1


def _gather_body(table_hbm, idx_hbm, out_hbm,
                 idx_all, rows0, rows1, tb0, tb1,
                 gsem0, gsem1, wsem0, wsem1):
    wid = lax.axis_index("s") * NUM_CORES + lax.axis_index("c")
    r0 = wid * ROWS_PER_W
    bufs = ((rows0, tb0, gsem0, wsem0), (rows1, tb1, gsem1, wsem1))

    pltpu.sync_copy(idx_hbm.at[pl.ds(r0, ROWS_PER_W)], idx_all)

    def stage(i_local, b):
        rows_v, _, gsem, _ = bufs[b]
        pltpu.async_copy(table_hbm.at[idx_all.at[i_local]], rows_v, gsem)

    iota = lax.iota(jnp.int32, 16)
    dvecs = tuple(iota + 16 * k for k in range(EMBED // 16))

    def wb_descs(i_local, b):
        _, tbuf, _, wsem = bufs[b]
        r = r0 + i_local
        h = r // ROWS_PER_H
        c = r % ROWS_PER_H
        return [(tbuf.at[pl.ds(8 * t, 8), pl.ds(0, ROW)],
                 out_hbm.at[h, t, c], wsem) for t in range(EMBED // 8)]

    def drain_wb(i_local, b):
        for src, dst, wsem in wb_descs(i_local, b):
            pltpu.make_async_copy(src, dst, wsem).wait()

    def finish(i_local, b):
        rows_v, tbuf, gsem, wsem = bufs[b]
        pltpu.make_async_copy(
            table_hbm.at[idx_all.at[i_local]], rows_v, gsem).wait()

        @pl.when(i_local >= 2)
        def _():
            drain_wb(i_local, b)

        @plsc.parallel_loop(0, ROW, step=1, unroll=4)
        def _(l):
            lsplat = jnp.full((16,), l, jnp.int32)
            for k in range(EMBED // 16):
                v = rows_v[l, pl.ds(16 * k, 16)]
                plsc.store_scatter(tbuf, [dvecs[k], lsplat], v)

        for src, dst, wsem_ in wb_descs(i_local, b):
            pltpu.async_copy(src, dst, wsem_)

    stage(0, 0)

    def body(i, carry):
        il = 2 * i
        stage(il + 1, 1)
        finish(il, 0)

        @pl.when(i < ROWS_PER_W // 2 - 1)
        def _():
            stage(il + 2, 0)

        finish(il + 1, 1)
        return carry

    lax.fori_loop(0, ROWS_PER_W // 2, body, 0)
    drain_wb(ROWS_PER_W - 2, 0)
    drain_wb(ROWS_PER_W - 1, 1)


@jax.jit
def kernel(x, table):
    idxT = (x.T.reshape(NROWS, ROW) * 2).astype(jnp.int32)
    tp = jnp.pad(table, ((0, 0), (0, TPAD - EMBED))).reshape(2 * NP, EMBED)
    mesh = plsc.VectorSubcoreMesh(core_axis_name="c", subcore_axis_name="s")
    out5d = pl.kernel(
        _gather_body,
        mesh=mesh,
        out_type=jax.ShapeDtypeStruct(
            (HIST, EMBED // 8, ROWS_PER_H, 8, ROW), jnp.float32),
        scratch_types=[
            pltpu.VMEM((ROWS_PER_W, ROW), jnp.int32),
            pltpu.VMEM((ROW, EMBED), jnp.float32),
            pltpu.VMEM((ROW, EMBED), jnp.float32),
            pltpu.VMEM((EMBED, PITCH), jnp.float32),
            pltpu.VMEM((EMBED, PITCH), jnp.float32),
            pltpu.SemaphoreType.DMA,
            pltpu.SemaphoreType.DMA,
            pltpu.SemaphoreType.DMA,
            pltpu.SemaphoreType.DMA,
        ],
        compiler_params=pltpu.CompilerParams(
            use_tc_tiling_on_sc=False, needs_layout_passes=False),
    )(tp, idxT)
    return out5d.transpose(2, 4, 0, 1, 3).reshape(BATCH, HIST, EMBED)
